# baseline (speedup 1.0000x reference)
import jax
import jax.numpy as jnp
from jax.experimental import pallas as pl
from jax.experimental.pallas import tpu as pltpu

B = 4096
BS = 1024
HBS = 512
NH = BS // HBS
NB = B // BS
NC = 3
K = 1024
D = 256
BETA = 0.25


def _rq_body(x_ref,
             ew0, ew1, ew2, ew3, eb0, eb1, eb2, eb3,
             dw0, dw1, dw2, dw3, db0, db1, db2, db3,
             cb_ref,
             out_ref, stats_ref, cn_ref, comp_ref):
    i = pl.program_id(0)

    def mm(a, w):
        return jax.lax.dot_general(
            a, w, (((1,), (0,)), ((), ())),
            preferred_element_type=jnp.float32)

    jidx = jax.lax.broadcasted_iota(jnp.int32, (HBS, K), 1)

    @pl.when(i == 0)
    def _cn():
        ones_row = jnp.ones((1, D), jnp.float32)
        for c in range(NC):
            cbq = cb_ref[c]
            cn_ref[c] = jax.lax.dot_general(
                ones_row, cbq * cbq, (((1,), (1,)), ((), ())),
                preferred_element_type=jnp.float32)

    def both(f, xs):
        return [f(v) for v in xs]

    hs = [x_ref[pl.ds(hfi * HBS, HBS), :] for hfi in range(NH)]
    xbs = hs
    for ew, eb in ((ew0, eb0), (ew1, eb1), (ew2, eb2), (ew3, eb3)):
        hs = both(lambda v: mm(v, ew[...]) + eb[...], hs)
        if ew is not ew3:
            hs = both(lambda v: jnp.maximum(v, 0.0), hs)
    ress = hs

    qsums = [jnp.zeros((HBS, 1), jnp.float32) for _ in range(NH)]
    z_hats = [jnp.zeros((HBS, D), jnp.float32) for _ in range(NH)]
    id_triples = [[] for _ in range(NH)]
    norm_cols = [[] for _ in range(NH)]
    for c in range(NC):
        cb = cb_ref[c]
        cn_row = cn_ref[c]
        dots = both(lambda v: jax.lax.dot_general(
            v, cb, (((1,), (1,)), ((), ())),
            preferred_element_type=jnp.float32), ress)
        ds_ = both(lambda v: cn_row - 2.0 * v, dots)
        dmins = both(lambda v: jnp.min(v, axis=1, keepdims=True), ds_)
        ids_cols = [jnp.min(jnp.where(ds_[t] == dmins[t], jidx, K), axis=1,
                            keepdims=True) for t in range(NH)]
        onehots = both(lambda v: (jidx == v).astype(jnp.float32), ids_cols)
        embs = both(lambda v: mm(v, cb), onehots)
        for t in range(NH):
            diff = ress[t] - embs[t]
            qsums[t] = qsums[t] + (BETA + 1.0) * jnp.sum(
                diff * diff, axis=1, keepdims=True)
            emb_st = ress[t] + (embs[t] - ress[t])
            norm_cols[t].append(jnp.sqrt(jnp.sum(emb_st * emb_st, axis=1,
                                                 keepdims=True)))
            z_hats[t] = z_hats[t] + emb_st
            id_triples[t].append(ids_cols[t])
            ress[t] = ress[t] - emb_st

    hs = z_hats
    for dw, db in ((dw0, db0), (dw1, db1), (dw2, db2), (dw3, db3)):
        hs = both(lambda v: mm(v, dw[...]) + db[...], hs)
        if dw is not dw3:
            hs = both(lambda v: jnp.maximum(v, 0.0), hs)
    x_hats = hs

    recon_blk = jnp.zeros((), jnp.float32)
    qloss_blk = jnp.zeros((), jnp.float32)
    ident = (jax.lax.broadcasted_iota(jnp.int32, (HBS, HBS), 0) ==
             jax.lax.broadcasted_iota(jnp.int32, (HBS, HBS), 1)
             ).astype(jnp.float32)
    for t in range(NH):
        sl = pl.ds(t * HBS, HBS)
        out_ref[0, sl, :] = jnp.concatenate(
            norm_cols[t] + [jnp.zeros((HBS, 13), jnp.float32)], axis=1)
        comps = []
        for idc in id_triples[t]:
            comps.append(jnp.bitwise_and(idc, 255).astype(jnp.float32))
            comps.append(jax.lax.shift_right_logical(idc, 8)
                         .astype(jnp.float32))
        a_mat = jnp.concatenate(comps, axis=1)
        at = jax.lax.dot_general(a_mat, ident, (((0,), (0,)), ((), ())),
                                 preferred_element_type=jnp.float32)
        comp_ref[i, 0:6, sl] = at
        r = x_hats[t] - xbs[t]
        recon_blk = recon_blk + jnp.sum(r * r)
        qloss_blk = qloss_blk + jnp.sum(qsums[t])

    @pl.when(i == 0)
    def _init():
        stats_ref[3:4, :] = jnp.full((1, 128), recon_blk, jnp.float32)
        stats_ref[4:5, :] = jnp.full((1, 128), qloss_blk, jnp.float32)

    @pl.when(i > 0)
    def _acc():
        stats_ref[3:4, :] = stats_ref[3:4, :] + recon_blk
        stats_ref[4:5, :] = stats_ref[4:5, :] + qloss_blk

    @pl.when(i == NB - 1)
    def _final():
        rs = stats_ref[3, 0]
        qs = stats_ref[4, 0]
        stats_ref[0:1, :] = jnp.full((1, 128), (rs + qs) / B, jnp.float32)
        stats_ref[1:2, :] = jnp.full((1, 128), rs / B, jnp.float32)
        stats_ref[2:3, :] = jnp.full((1, 128), qs / B, jnp.float32)

        def key_from(comp6):
            return comp6

        scale = [1, 256, K, 256 * K, K * K, 256 * K * K]
        krows = []
        for b in range(NB):
            kr = jnp.zeros((1, BS), jnp.int32)
            for ci in range(6):
                kr = kr + comp_ref[b, ci:ci + 1, :].astype(jnp.int32) *                     scale[ci]
            krows.append(kr)
        krow = jnp.concatenate(krows, axis=1)
        ident128 = (jax.lax.broadcasted_iota(jnp.int32, (128, 128), 0) ==
                    jax.lax.broadcasted_iota(jnp.int32, (128, 128), 1)
                    ).astype(jnp.float32)
        inv_sum = jnp.zeros((), jnp.float32)
        for rch in range(B // 128):
            bidx, off = rch // (BS // 128), (rch % (BS // 128)) * 128
            chunk = comp_ref[bidx, 0:6, pl.ds(off, 128)]
            colt = jax.lax.dot_general(
                ident128, chunk, (((1,), (1,)), ((), ())),
                preferred_element_type=jnp.float32)
            ck = jnp.zeros((128, 1), jnp.int32)
            for ci in range(6):
                ck = ck + colt[:, ci:ci + 1].astype(jnp.int32) * scale[ci]
            cnt = jnp.sum(jnp.where(ck == krow, 1.0, 0.0), axis=1)
            inv_sum = inv_sum + jnp.sum(1.0 / cnt)
        stats_ref[5:6, :] = jnp.full((1, 128), jnp.round(inv_sum) / B,
                                     jnp.float32)


def kernel(x, enc_Ws, enc_bs, dec_Ws, dec_bs, codebooks, temperature):
    del temperature
    enc_bs = [b.reshape(1, -1) for b in enc_bs]
    dec_bs = [b.reshape(1, -1) for b in dec_bs]

    full = lambda a: pl.BlockSpec(a.shape, lambda i: (0,) * a.ndim)
    in_specs = [pl.BlockSpec((BS, 768), lambda i: (i, 0))]
    in_specs += [full(w) for w in enc_Ws] + [full(b) for b in enc_bs]
    in_specs += [full(w) for w in dec_Ws] + [full(b) for b in dec_bs]
    in_specs += [full(codebooks)]

    packed, stats = pl.pallas_call(
        _rq_body,
        grid=(NB,),
        in_specs=in_specs,
        out_specs=[
            pl.BlockSpec((1, BS, 16), lambda i: (i, 0, 0)),
            pl.BlockSpec((8, 128), lambda i: (0, 0)),
        ],
        out_shape=[
            jax.ShapeDtypeStruct((NB, BS, 16), jnp.float32),
            jax.ShapeDtypeStruct((8, 128), jnp.float32),
        ],
        scratch_shapes=[pltpu.VMEM((NC, 1, K), jnp.float32),
                        pltpu.VMEM((NB, 8, BS), jnp.float32)],
        compiler_params=pltpu.CompilerParams(
            dimension_semantics=("arbitrary",)),
    )(x, *enc_Ws, *enc_bs, *dec_Ws, *dec_bs, codebooks)

    embs_norm = packed[:, :, :NC].reshape(B, NC)
    loss = stats[0, 0]
    mean_recon = stats[1, 0]
    mean_qloss = stats[2, 0]
    return (loss, mean_recon, mean_qloss, embs_norm, stats[5, 0])


# speedup vs baseline: 1.2003x; 1.2003x over previous
import jax
import jax.numpy as jnp
from jax.experimental import pallas as pl
from jax.experimental.pallas import tpu as pltpu

B = 4096
BS = 1024
HBS = 512
NH = BS // HBS
NB = B // BS
NC = 3
K = 1024
D = 256
BETA = 0.25


def _rq_body(x_ref,
             ew0, ew1, ew2, ew3, eb0, eb1, eb2, eb3,
             dw0, dw1, dw2, dw3, db0, db1, db2, db3,
             cb_ref,
             out_ref, stats_ref, cn_ref):
    i = pl.program_id(0)

    def mm(a, w):
        return jax.lax.dot_general(
            a, w, (((1,), (0,)), ((), ())),
            preferred_element_type=jnp.float32)

    jidx = jax.lax.broadcasted_iota(jnp.int32, (HBS, K), 1)

    @pl.when(i == 0)
    def _cn():
        ones_row = jnp.ones((1, D), jnp.float32)
        for c in range(NC):
            cbq = cb_ref[c]
            cn_ref[c] = jax.lax.dot_general(
                ones_row, cbq * cbq, (((1,), (1,)), ((), ())),
                preferred_element_type=jnp.float32)

    def both(f, xs):
        return [f(v) for v in xs]

    hs = [x_ref[pl.ds(hfi * HBS, HBS), :] for hfi in range(NH)]
    xbs = hs
    for ew, eb in ((ew0, eb0), (ew1, eb1), (ew2, eb2), (ew3, eb3)):
        hs = both(lambda v: mm(v, ew[...]) + eb[...], hs)
        if ew is not ew3:
            hs = both(lambda v: jnp.maximum(v, 0.0), hs)
    ress = hs

    qsums = [jnp.zeros((HBS, 1), jnp.float32) for _ in range(NH)]
    z_hats = [jnp.zeros((HBS, D), jnp.float32) for _ in range(NH)]
    key_cols = [jnp.zeros((HBS, 1), jnp.int32) for _ in range(NH)]
    norm_cols = [[] for _ in range(NH)]
    for c in range(NC):
        cb = cb_ref[c]
        cn_row = cn_ref[c]
        dots = both(lambda v: jax.lax.dot_general(
            v, cb, (((1,), (1,)), ((), ())),
            preferred_element_type=jnp.float32), ress)
        ds_ = both(lambda v: cn_row - 2.0 * v, dots)
        dmins = both(lambda v: jnp.min(v, axis=1, keepdims=True), ds_)
        ids_cols = [jnp.min(jnp.where(ds_[t] == dmins[t], jidx, K), axis=1,
                            keepdims=True) for t in range(NH)]
        onehots = both(lambda v: (jidx == v).astype(jnp.float32), ids_cols)
        embs = both(lambda v: mm(v, cb), onehots)
        for t in range(NH):
            diff = ress[t] - embs[t]
            qsums[t] = qsums[t] + (BETA + 1.0) * jnp.sum(
                diff * diff, axis=1, keepdims=True)
            emb_st = ress[t] + (embs[t] - ress[t])
            norm_cols[t].append(jnp.sqrt(jnp.sum(emb_st * emb_st, axis=1,
                                                 keepdims=True)))
            z_hats[t] = z_hats[t] + emb_st
            key_cols[t] = key_cols[t] + ids_cols[t] * (K ** c)
            ress[t] = ress[t] - emb_st

    hs = z_hats
    for dw, db in ((dw0, db0), (dw1, db1), (dw2, db2), (dw3, db3)):
        hs = both(lambda v: mm(v, dw[...]) + db[...], hs)
        if dw is not dw3:
            hs = both(lambda v: jnp.maximum(v, 0.0), hs)
    x_hats = hs

    recon_blk = jnp.zeros((), jnp.float32)
    qloss_blk = jnp.zeros((), jnp.float32)
    for t in range(NH):
        sl = pl.ds(t * HBS, HBS)
        keyf = jax.lax.bitcast_convert_type(key_cols[t], jnp.float32)
        out_ref[0, sl, :] = jnp.concatenate(
            norm_cols[t] + [keyf, jnp.zeros((HBS, 12), jnp.float32)], axis=1)
        r = x_hats[t] - xbs[t]
        recon_blk = recon_blk + jnp.sum(r * r)
        qloss_blk = qloss_blk + jnp.sum(qsums[t])

    keep = jnp.where(i > 0, 1.0, 0.0)
    stats_ref[3:4, :] = keep * stats_ref[3:4, :] + recon_blk
    stats_ref[4:5, :] = keep * stats_ref[4:5, :] + qloss_blk

    @pl.when(i == NB - 1)
    def _final():
        rs = stats_ref[3, 0]
        qs = stats_ref[4, 0]
        stats_ref[0:1, :] = jnp.full((1, 128), (rs + qs) / B, jnp.float32)
        stats_ref[1:2, :] = jnp.full((1, 128), rs / B, jnp.float32)
        stats_ref[2:3, :] = jnp.full((1, 128), qs / B, jnp.float32)


def _unique_body(krow_ref, kmat_ref, out_ref):
    krow = krow_ref[...]
    kt = jnp.transpose(kmat_ref[...])
    inv_sum = jnp.zeros((), jnp.float32)
    for r in range(B // 128):
        a = kt[:, r:r + 1]
        cnt = jnp.sum(jnp.where(a == krow, 1.0, 0.0), axis=1)
        inv_sum = inv_sum + jnp.sum(1.0 / cnt)
    p = jnp.round(inv_sum) / B
    out_ref[...] = jnp.full((1, 128), p, jnp.float32)


def kernel(x, enc_Ws, enc_bs, dec_Ws, dec_bs, codebooks, temperature):
    del temperature
    enc_bs = [b.reshape(1, -1) for b in enc_bs]
    dec_bs = [b.reshape(1, -1) for b in dec_bs]

    full = lambda a: pl.BlockSpec(a.shape, lambda i: (0,) * a.ndim)
    in_specs = [pl.BlockSpec((BS, 768), lambda i: (i, 0))]
    in_specs += [full(w) for w in enc_Ws] + [full(b) for b in enc_bs]
    in_specs += [full(w) for w in dec_Ws] + [full(b) for b in dec_bs]
    in_specs += [full(codebooks)]

    packed, stats = pl.pallas_call(
        _rq_body,
        grid=(NB,),
        in_specs=in_specs,
        out_specs=[
            pl.BlockSpec((1, BS, 16), lambda i: (i, 0, 0)),
            pl.BlockSpec((8, 128), lambda i: (0, 0)),
        ],
        out_shape=[
            jax.ShapeDtypeStruct((NB, BS, 16), jnp.float32),
            jax.ShapeDtypeStruct((8, 128), jnp.float32),
        ],
        scratch_shapes=[pltpu.VMEM((NC, 1, K), jnp.float32)],
        compiler_params=pltpu.CompilerParams(
            dimension_semantics=("arbitrary",)),
    )(x, *enc_Ws, *enc_bs, *dec_Ws, *dec_bs, codebooks)

    keys = jax.lax.bitcast_convert_type(packed[:, :, 3], jnp.int32)
    krow = keys.reshape(1, B)
    kmat = keys.reshape(B // 128, 128)
    p_unique = pl.pallas_call(
        _unique_body,
        in_specs=[pl.BlockSpec((1, B), lambda: (0, 0)),
                  pl.BlockSpec((B // 128, 128), lambda: (0, 0))],
        out_specs=pl.BlockSpec((1, 128), lambda: (0, 0)),
        out_shape=jax.ShapeDtypeStruct((1, 128), jnp.float32),
    )(krow, kmat)

    embs_norm = packed[:, :, :NC].reshape(B, NC)
    loss = stats[0, 0]
    mean_recon = stats[1, 0]
    mean_qloss = stats[2, 0]
    return (loss, mean_recon, mean_qloss, embs_norm, p_unique[0, 0])
